# baseline (speedup 1.0000x reference)
import jax
import jax.numpy as jnp
from jax.experimental import pallas as pl
from jax.experimental.pallas import tpu as pltpu

N_TOKENS = 8192
HIDDEN = 2048
OUT = 2048
N_EXPERTS = 8
BN = 1024
BK = 512


def _moe_body(x_ref, wg_ref, bg_ref, wu_ref, bu_ref, wv_ref, bv_ref,
              wo_ref, bo_ref, out_ref, gates_scr):
    e = pl.program_id(1)
    k = pl.program_id(2)
    x = x_ref[...]

    @pl.when(jnp.logical_and(e == 0, k == 0))
    def _():
        logits = jnp.dot(x, wg_ref[...], preferred_element_type=jnp.float32)
        logits = logits + bg_ref[...][None, :]
        m = jnp.max(logits, axis=-1, keepdims=True)
        ex = jnp.exp(logits - m)
        gates_scr[...] = ex / jnp.sum(ex, axis=-1, keepdims=True)

    u = jnp.dot(x, wu_ref[0], preferred_element_type=jnp.float32) + bu_ref[0]
    v = jnp.dot(x, wv_ref[0], preferred_element_type=jnp.float32) + bv_ref[0]
    h = (u * jax.nn.sigmoid(u)) * v
    part = jnp.dot(h.astype(jnp.bfloat16), wo_ref[0],
                   preferred_element_type=jnp.float32)

    lane = jax.lax.broadcasted_iota(jnp.int32, (1, N_EXPERTS), 1)
    g = jnp.sum(jnp.where(lane == e, gates_scr[...], 0.0), axis=-1,
                keepdims=True)
    part = part + jnp.where(k == 0, 1.0, 0.0) * bo_ref[0]
    contrib = g * part

    first = jnp.logical_and(e == 0, k == 0)

    @pl.when(first)
    def _():
        out_ref[...] = contrib

    @pl.when(jnp.logical_not(first))
    def _():
        out_ref[...] = out_ref[...] + contrib


@jax.jit
def kernel(inputs, Wg, bg, Wu, bu, Wv, bv, Wo, bo):
    grid = (N_TOKENS // BN, N_EXPERTS, HIDDEN // BK)
    x16 = inputs.astype(jnp.bfloat16)
    Wg16 = Wg.astype(jnp.bfloat16)
    Wu16 = Wu.astype(jnp.bfloat16)
    Wv16 = Wv.astype(jnp.bfloat16)
    Wo16 = Wo.astype(jnp.bfloat16)
    bu = bu.reshape(N_EXPERTS, 1, HIDDEN)
    bv = bv.reshape(N_EXPERTS, 1, HIDDEN)
    bo = bo.reshape(N_EXPERTS, 1, OUT)
    return pl.pallas_call(
        _moe_body,
        grid=grid,
        in_specs=[
            pl.BlockSpec((BN, HIDDEN), lambda i, e, k: (i, 0)),
            pl.BlockSpec((HIDDEN, N_EXPERTS), lambda i, e, k: (0, 0)),
            pl.BlockSpec((N_EXPERTS,), lambda i, e, k: (0,)),
            pl.BlockSpec((1, HIDDEN, BK), lambda i, e, k: (e, 0, k)),
            pl.BlockSpec((1, 1, BK), lambda i, e, k: (e, 0, k)),
            pl.BlockSpec((1, HIDDEN, BK), lambda i, e, k: (e, 0, k)),
            pl.BlockSpec((1, 1, BK), lambda i, e, k: (e, 0, k)),
            pl.BlockSpec((1, BK, OUT), lambda i, e, k: (e, k, 0)),
            pl.BlockSpec((1, 1, OUT), lambda i, e, k: (e, 0, 0)),
        ],
        out_specs=pl.BlockSpec((BN, OUT), lambda i, e, k: (i, 0)),
        out_shape=jax.ShapeDtypeStruct((N_TOKENS, OUT), jnp.float32),
        scratch_shapes=[pltpu.VMEM((BN, N_EXPERTS), jnp.float32)],
        compiler_params=pltpu.CompilerParams(
            dimension_semantics=("arbitrary", "arbitrary", "arbitrary"),
        ),
    )(x16, Wg16, bg, Wu16, bu, Wv16, bv, Wo16, bo)


# speedup vs baseline: 1.1004x; 1.1004x over previous
import jax
import jax.numpy as jnp
from jax.experimental import pallas as pl
from jax.experimental.pallas import tpu as pltpu

N_TOKENS = 8192
HIDDEN = 2048
OUT = 2048
N_EXPERTS = 8
BN = 1024
BK = 1024


def _moe_body(x_ref, wg_ref, bg_ref, wu_ref, bu_ref, wv_ref, bv_ref,
              wo_ref, bo_ref, out_ref, gates_scr):
    e = pl.program_id(1)
    k = pl.program_id(2)
    x = x_ref[...]

    @pl.when(jnp.logical_and(e == 0, k == 0))
    def _():
        logits = jnp.dot(x, wg_ref[...], preferred_element_type=jnp.float32)
        logits = logits + bg_ref[...][None, :]
        m = jnp.max(logits, axis=-1, keepdims=True)
        ex = jnp.exp(logits - m)
        gates_scr[...] = ex / jnp.sum(ex, axis=-1, keepdims=True)

    u = jnp.dot(x, wu_ref[0], preferred_element_type=jnp.float32) + bu_ref[0]
    v = jnp.dot(x, wv_ref[0], preferred_element_type=jnp.float32) + bv_ref[0]

    lane = jax.lax.broadcasted_iota(jnp.int32, (1, N_EXPERTS), 1)
    g = jnp.sum(jnp.where(lane == e, gates_scr[...], 0.0), axis=-1,
                keepdims=True)
    h = ((u * jax.nn.sigmoid(u)) * v) * g
    part = jnp.dot(h.astype(jnp.bfloat16), wo_ref[0],
                   preferred_element_type=jnp.float32)
    contrib = part + jnp.where(k == 0, 1.0, 0.0) * (g * bo_ref[0])

    first = jnp.logical_and(e == 0, k == 0)

    @pl.when(first)
    def _():
        out_ref[...] = contrib

    @pl.when(jnp.logical_not(first))
    def _():
        out_ref[...] = out_ref[...] + contrib


@jax.jit
def kernel(inputs, Wg, bg, Wu, bu, Wv, bv, Wo, bo):
    grid = (N_TOKENS // BN, N_EXPERTS, HIDDEN // BK)
    x16 = inputs.astype(jnp.bfloat16)
    Wg16 = Wg.astype(jnp.bfloat16)
    Wu16 = Wu.astype(jnp.bfloat16)
    Wv16 = Wv.astype(jnp.bfloat16)
    Wo16 = Wo.astype(jnp.bfloat16)
    bu = bu.reshape(N_EXPERTS, 1, HIDDEN)
    bv = bv.reshape(N_EXPERTS, 1, HIDDEN)
    bo = bo.reshape(N_EXPERTS, 1, OUT)
    return pl.pallas_call(
        _moe_body,
        grid=grid,
        in_specs=[
            pl.BlockSpec((BN, HIDDEN), lambda i, e, k: (i, 0)),
            pl.BlockSpec((HIDDEN, N_EXPERTS), lambda i, e, k: (0, 0)),
            pl.BlockSpec((N_EXPERTS,), lambda i, e, k: (0,)),
            pl.BlockSpec((1, HIDDEN, BK), lambda i, e, k: (e, 0, k)),
            pl.BlockSpec((1, 1, BK), lambda i, e, k: (e, 0, k)),
            pl.BlockSpec((1, HIDDEN, BK), lambda i, e, k: (e, 0, k)),
            pl.BlockSpec((1, 1, BK), lambda i, e, k: (e, 0, k)),
            pl.BlockSpec((1, BK, OUT), lambda i, e, k: (e, k, 0)),
            pl.BlockSpec((1, 1, OUT), lambda i, e, k: (e, 0, 0)),
        ],
        out_specs=pl.BlockSpec((BN, OUT), lambda i, e, k: (i, 0)),
        out_shape=jax.ShapeDtypeStruct((N_TOKENS, OUT), jnp.float32),
        scratch_shapes=[pltpu.VMEM((BN, N_EXPERTS), jnp.float32)],
        compiler_params=pltpu.CompilerParams(
            dimension_semantics=("arbitrary", "arbitrary", "arbitrary"),
        ),
    )(x16, Wg16, bg, Wu16, bu, Wv16, bv, Wo16, bo)
